# baseline (speedup 1.0000x reference)
import functools

import jax
import jax.numpy as jnp
from jax import lax
from jax.experimental import pallas as pl
from jax.experimental.pallas import tpu as pltpu
from jax.experimental.pallas import tpu_sc as plsc

VOCAB = 100000
EMBED = 1024
WINDOW = 2048
BATCH = 4

NUM_CORES = 2
NUM_SUBCORES = 16
NUM_WORKERS = NUM_CORES * NUM_SUBCORES
POS_PER_WORKER = WINDOW // NUM_WORKERS
CHUNK = 8
NCHUNK = POS_PER_WORKER // CHUNK
LANES = 16
VECS_PER_ROW = EMBED // LANES


def _make_kernel():
    mesh = plsc.VectorSubcoreMesh(core_axis_name="c", subcore_axis_name="s")

    @functools.partial(
        pl.kernel,
        mesh=mesh,
        out_type=jax.ShapeDtypeStruct((BATCH * WINDOW, EMBED), jnp.float32),
        scratch_types=[
            pltpu.VMEM((BATCH * POS_PER_WORKER,), jnp.int32),
            pltpu.VMEM((3, CHUNK, EMBED), jnp.float32),
            pltpu.VMEM((3, BATCH, CHUNK, EMBED), jnp.float32),
            pltpu.SemaphoreType.DMA,
            pltpu.SemaphoreType.DMA,
            pltpu.SemaphoreType.DMA,
            pltpu.SemaphoreType.DMA,
            pltpu.SemaphoreType.DMA,
            pltpu.SemaphoreType.DMA,
            pltpu.SemaphoreType.DMA,
        ],
    )
    def emb_kernel(tokens_hbm, ttab_hbm, ptab_hbm, out_hbm,
                   idx_v, pos_v, tok_v,
                   gsem0, gsem1, gsem2, ssem0, ssem1, ssem2, psem):
        wid = lax.axis_index("s") * NUM_CORES + lax.axis_index("c")
        pstart = wid * POS_PER_WORKER
        gsems = (gsem0, gsem1, gsem2)
        ssems = (ssem0, ssem1, ssem2)

        for b in range(BATCH):
            pltpu.sync_copy(
                tokens_hbm.at[pl.ds(b * WINDOW + pstart, POS_PER_WORKER)],
                idx_v.at[pl.ds(b * POS_PER_WORKER, POS_PER_WORKER)])

        NBUF = 3
        JUNROLL = 32
        NJB = VECS_PER_ROW // JUNROLL

        def issue(c):
            s = c % NBUF
            cps = [pltpu.async_copy(
                ptab_hbm.at[pl.ds(pstart + c * CHUNK, CHUNK)],
                pos_v.at[s], psem)]
            for b in range(BATCH):
                idx_sl = idx_v.at[pl.ds(b * POS_PER_WORKER + c * CHUNK, CHUNK)]
                cps.append(pltpu.async_copy(
                    ttab_hbm.at[idx_sl], tok_v.at[s, b], gsems[s]))
            return cps

        pending_in = {0: issue(0), 1: issue(1)}
        pending_st = {}
        for c in range(NCHUNK):
            s = c % NBUF
            if c + 2 < NCHUNK:
                if c - 1 in pending_st:
                    for cp in pending_st.pop(c - 1):
                        cp.wait()
                pending_in[c + 2] = issue(c + 2)
            for cp in pending_in.pop(c):
                cp.wait()

            def body(i, carry):
                r = i // NJB
                jb = i % NJB
                for jj in range(JUNROLL):
                    sl = pl.ds(jb * JUNROLL * LANES + jj * LANES, LANES)
                    p = pos_v[s, r, sl]
                    for b in range(BATCH):
                        tok_v[s, b, r, sl] = tok_v[s, b, r, sl] + p
                return carry

            lax.fori_loop(0, CHUNK * NJB, body, 0)

            sts = []
            for b in range(BATCH):
                sts.append(pltpu.async_copy(
                    tok_v.at[s, b],
                    out_hbm.at[pl.ds(b * WINDOW + pstart + c * CHUNK, CHUNK)],
                    ssems[s]))
            pending_st[c] = sts
        for sts in pending_st.values():
            for cp in sts:
                cp.wait()

    return emb_kernel


_EMB_KERNEL = _make_kernel()


def kernel(tokens, token_table, position_table):
    flat_tokens = tokens.reshape(BATCH * WINDOW).astype(jnp.int32)
    out = _EMB_KERNEL(flat_tokens, token_table, position_table)
    return out.reshape(BATCH, WINDOW, EMBED)


# speedup vs baseline: 1.1189x; 1.1189x over previous
import functools

import jax
import jax.numpy as jnp
from jax import lax
from jax.experimental import pallas as pl
from jax.experimental.pallas import tpu as pltpu
from jax.experimental.pallas import tpu_sc as plsc

VOCAB = 100000
EMBED = 1024
WINDOW = 2048
BATCH = 4

NUM_CORES = 2
NUM_SUBCORES = 16
NUM_WORKERS = NUM_CORES * NUM_SUBCORES
POS_PER_WORKER = WINDOW // NUM_WORKERS
CHUNK = 8
NCHUNK = POS_PER_WORKER // CHUNK
LANES = 16
VECS_PER_ROW = EMBED // LANES


def _make_kernel():
    mesh = plsc.VectorSubcoreMesh(core_axis_name="c", subcore_axis_name="s")

    @functools.partial(
        pl.kernel,
        mesh=mesh,
        out_type=jax.ShapeDtypeStruct((BATCH * WINDOW, EMBED), jnp.float32),
        scratch_types=[
            pltpu.VMEM((BATCH * POS_PER_WORKER,), jnp.int32),
            pltpu.VMEM((3, CHUNK, EMBED), jnp.float32),
            pltpu.VMEM((3, BATCH, CHUNK, EMBED), jnp.float32),
            pltpu.SemaphoreType.DMA,
            pltpu.SemaphoreType.DMA,
            pltpu.SemaphoreType.DMA,
            pltpu.SemaphoreType.DMA,
            pltpu.SemaphoreType.DMA,
            pltpu.SemaphoreType.DMA,
            pltpu.SemaphoreType.DMA,
        ],
    )
    def emb_kernel(tokens_hbm, ttab_hbm, ptab_hbm, out_hbm,
                   idx_v, pos_v, tok_v,
                   gsem0, gsem1, gsem2, ssem0, ssem1, ssem2, psem):
        wid = lax.axis_index("s") * NUM_CORES + lax.axis_index("c")
        pstart = wid * POS_PER_WORKER
        gsems = (gsem0, gsem1, gsem2)
        ssems = (ssem0, ssem1, ssem2)

        for b in range(BATCH):
            pltpu.sync_copy(
                tokens_hbm.at[pl.ds(b * WINDOW + pstart, POS_PER_WORKER)],
                idx_v.at[pl.ds(b * POS_PER_WORKER, POS_PER_WORKER)])

        NBUF = 3
        JUNROLL = 32
        NJB = VECS_PER_ROW // JUNROLL

        def issue(c):
            s = c % NBUF
            cps = [pltpu.async_copy(
                ptab_hbm.at[pl.ds(pstart + c * CHUNK, CHUNK)],
                pos_v.at[s], psem)]
            for b in range(BATCH):
                idx_sl = idx_v.at[pl.ds(b * POS_PER_WORKER + c * CHUNK, CHUNK)]
                cps.append(pltpu.async_copy(
                    ttab_hbm.at[idx_sl], tok_v.at[s, b], gsems[s]))
            return cps

        pending_in = {0: issue(0), 1: issue(1)}
        pending_st = {}
        for c in range(NCHUNK):
            s = c % NBUF
            if c + 2 < NCHUNK:
                if c - 1 in pending_st:
                    for cp in pending_st.pop(c - 1):
                        cp.wait()
                pending_in[c + 2] = issue(c + 2)
            for cp in pending_in.pop(c):
                cp.wait()

            def body(r, carry):
                for j in range(VECS_PER_ROW):
                    sl = pl.ds(j * LANES, LANES)
                    p = pos_v[s, r, sl]
                    for b in range(BATCH):
                        tok_v[s, b, r, sl] = tok_v[s, b, r, sl] + p
                return carry

            lax.fori_loop(0, CHUNK, body, 0)

            sts = []
            for b in range(BATCH):
                sts.append(pltpu.async_copy(
                    tok_v.at[s, b],
                    out_hbm.at[pl.ds(b * WINDOW + pstart + c * CHUNK, CHUNK)],
                    ssems[s]))
            pending_st[c] = sts
        for sts in pending_st.values():
            for cp in sts:
                cp.wait()

    return emb_kernel


_EMB_KERNEL = _make_kernel()


def kernel(tokens, token_table, position_table):
    flat_tokens = tokens.reshape(BATCH * WINDOW).astype(jnp.int32)
    out = _EMB_KERNEL(flat_tokens, token_table, position_table)
    return out.reshape(BATCH, WINDOW, EMBED)


# speedup vs baseline: 1.1193x; 1.0003x over previous
import functools

import jax
import jax.numpy as jnp
from jax import lax
from jax.experimental import pallas as pl
from jax.experimental.pallas import tpu as pltpu
from jax.experimental.pallas import tpu_sc as plsc

VOCAB = 100000
EMBED = 1024
WINDOW = 2048
BATCH = 4

NUM_CORES = 2
NUM_SUBCORES = 16
NUM_WORKERS = NUM_CORES * NUM_SUBCORES
POS_PER_WORKER = WINDOW // NUM_WORKERS
CHUNK = 8
NCHUNK = POS_PER_WORKER // CHUNK
LANES = 16
VECS_PER_ROW = EMBED // LANES
NBUF = 3


def _make_kernel():
    mesh = plsc.VectorSubcoreMesh(core_axis_name="c", subcore_axis_name="s")

    @functools.partial(
        pl.kernel,
        mesh=mesh,
        out_type=jax.ShapeDtypeStruct((BATCH, WINDOW, EMBED), jnp.float32),
        scratch_types=[
            pltpu.VMEM((BATCH * POS_PER_WORKER,), jnp.int32),
            pltpu.VMEM((NBUF, CHUNK, EMBED), jnp.float32),
            pltpu.VMEM((NBUF, BATCH, CHUNK, EMBED), jnp.float32),
            pltpu.SemaphoreType.DMA,
            pltpu.SemaphoreType.DMA,
            pltpu.SemaphoreType.DMA,
            pltpu.SemaphoreType.DMA,
            pltpu.SemaphoreType.DMA,
            pltpu.SemaphoreType.DMA,
            pltpu.SemaphoreType.DMA,
        ],
    )
    def emb_kernel(tokens_hbm, ttab_hbm, ptab_hbm, out_hbm,
                   idx_v, pos_v, tok_v,
                   gsem0, gsem1, gsem2, ssem0, ssem1, ssem2, psem):
        wid = lax.axis_index("s") * NUM_CORES + lax.axis_index("c")
        pstart = wid * POS_PER_WORKER
        gsems = (gsem0, gsem1, gsem2)
        ssems = (ssem0, ssem1, ssem2)

        for b in range(BATCH):
            pltpu.sync_copy(
                tokens_hbm.at[pl.ds(b * WINDOW + pstart, POS_PER_WORKER)],
                idx_v.at[pl.ds(b * POS_PER_WORKER, POS_PER_WORKER)])

        def issue(c):
            s = c % NBUF
            cps = [pltpu.async_copy(
                ptab_hbm.at[pl.ds(pstart + c * CHUNK, CHUNK)],
                pos_v.at[s], psem)]
            for b in range(BATCH):
                idx_sl = idx_v.at[pl.ds(b * POS_PER_WORKER + c * CHUNK, CHUNK)]
                cps.append(pltpu.async_copy(
                    ttab_hbm.at[idx_sl], tok_v.at[s, b], gsems[s]))
            return cps

        pending_in = {0: issue(0), 1: issue(1)}
        pending_st = {}
        for c in range(NCHUNK):
            s = c % NBUF
            if c + 2 < NCHUNK:
                if c - 1 in pending_st:
                    pending_st.pop(c - 1).wait()
                pending_in[c + 2] = issue(c + 2)
            for cp in pending_in.pop(c):
                cp.wait()

            def body(r, carry):
                for j in range(VECS_PER_ROW):
                    sl = pl.ds(j * LANES, LANES)
                    p = pos_v[s, r, sl]
                    for b in range(BATCH):
                        tok_v[s, b, r, sl] = tok_v[s, b, r, sl] + p
                return carry

            lax.fori_loop(0, CHUNK, body, 0)

            pending_st[c] = pltpu.async_copy(
                tok_v.at[s],
                out_hbm.at[:, pl.ds(pstart + c * CHUNK, CHUNK), :],
                ssems[s])
        for cp in pending_st.values():
            cp.wait()

    return emb_kernel


_EMB_KERNEL = _make_kernel()


def kernel(tokens, token_table, position_table):
    flat_tokens = tokens.reshape(BATCH * WINDOW).astype(jnp.int32)
    return _EMB_KERNEL(flat_tokens, token_table, position_table)
